# baseline (speedup 1.0000x reference)
import functools

import jax
import jax.numpy as jnp
from jax import lax
from jax.experimental import pallas as pl
from jax.experimental.pallas import tpu as pltpu
from jax.experimental.pallas import tpu_sc as plsc

EPS = 0.1
INT_MIN = -(2 ** 31)


def _mm_body(x_ref, w_ref, o_ref):
    o_ref[...] = jnp.dot(x_ref[...], w_ref[...],
                         preferred_element_type=jnp.float32)


def _proj(features, wcat):
    N, D = features.shape
    F = wcat.shape[1]
    bm = 1024
    return pl.pallas_call(
        _mm_body,
        grid=(N // bm,),
        in_specs=[pl.BlockSpec((bm, D), lambda i: (i, 0)),
                  pl.BlockSpec((D, F), lambda i: (0, 0))],
        out_specs=pl.BlockSpec((bm, F), lambda i: (i, 0)),
        out_shape=jax.ShapeDtypeStruct((N, F), jnp.float32),
    )(features, wcat)


def _big_body(adj_ref, fw_ref, s_ref, nv_ref, y_ref):
    adj = adj_ref[...]
    p = jnp.dot(adj.astype(jnp.bfloat16), fw_ref[...].astype(jnp.bfloat16),
                preferred_element_type=jnp.float32)
    nv_ref[...] = jax.nn.relu(p)
    y_ref[...] = jnp.dot(adj, s_ref[...], preferred_element_type=jnp.float32)


def _big(ori_adj, fw, s_col):
    N = ori_adj.shape[0]
    O = fw.shape[1]
    bm = 512
    return pl.pallas_call(
        _big_body,
        grid=(N // bm,),
        in_specs=[pl.BlockSpec((bm, N), lambda i: (i, 0)),
                  pl.BlockSpec((N, O), lambda i: (0, 0)),
                  pl.BlockSpec((N, 1), lambda i: (0, 0))],
        out_specs=[pl.BlockSpec((bm, O), lambda i: (i, 0)),
                   pl.BlockSpec((bm, 1), lambda i: (i, 0))],
        out_shape=[jax.ShapeDtypeStruct((N, O), jnp.float32),
                   jax.ShapeDtypeStruct((N, 1), jnp.float32)],
    )(ori_adj, fw, s_col)


def _mv_body(adj_ref, s_ref, m_ref, o_ref):
    sm = s_ref[...] * m_ref[...]
    o_ref[...] = jnp.dot(adj_ref[...], sm, preferred_element_type=jnp.float32)


def _masked_matvec(ori_adj, s_col, m_col):
    N = ori_adj.shape[0]
    bm = 512
    return pl.pallas_call(
        _mv_body,
        grid=(N // bm,),
        in_specs=[pl.BlockSpec((bm, N), lambda i: (i, 0)),
                  pl.BlockSpec((N, 1), lambda i: (0, 0)),
                  pl.BlockSpec((N, 1), lambda i: (0, 0))],
        out_specs=pl.BlockSpec((bm, 1), lambda i: (i, 0)),
        out_shape=jax.ShapeDtypeStruct((N, 1), jnp.float32),
    )(ori_adj, s_col, m_col)


def _greedy_max(key, k, steps):
    prefix = jnp.int32(0)
    for shift, tmax in steps:
        rows = []
        for t in range(1, tmax + 1):
            cand = prefix | jnp.int32(t << shift)
            rows.append(jnp.sum((key >= cand).astype(jnp.int32), axis=0,
                                keepdims=True))
        counts = jnp.sum(jnp.concatenate(rows, axis=0), axis=1)
        nib = jnp.sum((counts >= k).astype(jnp.int32))
        prefix = prefix | lax.shift_left(nib, jnp.int32(shift))
    return prefix


_VAL_STEPS = ((28, 3), (24, 15), (20, 15), (16, 15), (12, 15), (8, 15),
              (4, 15), (0, 15))
_IDX_STEPS = ((12, 1), (8, 15), (4, 15), (0, 15))


def _select_body(k, v_ref, prev_ref, o_ref):
    v = v_ref[...]
    bits = lax.bitcast_convert_type(v, jnp.int32)
    key = jnp.where(prev_ref[...] > 0.5, bits, jnp.int32(-1))
    kk = jnp.int32(k)
    kth = _greedy_max(key, kk, _VAL_STEPS)
    chigh = jnp.sum((key > kth).astype(jnp.int32))
    r = kk - chigh
    tie = key == kth
    rr = lax.broadcasted_iota(jnp.int32, v.shape, 0)
    cc = lax.broadcasted_iota(jnp.int32, v.shape, 1)
    nel = v.shape[0] * v.shape[1]
    key2 = jnp.where(tie, jnp.int32(nel) - (rr * v.shape[1] + cc),
                     jnp.int32(0))
    kth2 = _greedy_max(key2, r, _IDX_STEPS)
    sel = (key > kth) | (key2 >= kth2)
    o_ref[...] = sel.astype(jnp.float32)


def _topk_mask(v, prev, k):
    R, C = v.shape
    return pl.pallas_call(
        functools.partial(_select_body, k),
        in_specs=[pl.BlockSpec((R, C), lambda: (0, 0)),
                  pl.BlockSpec((R, C), lambda: (0, 0))],
        out_specs=pl.BlockSpec((R, C), lambda: (0, 0)),
        out_shape=jax.ShapeDtypeStruct((R, C), jnp.float32),
    )(v, prev)


def _zn_body(nb, d, x_ref, w_ref, o_ref):
    x = x_ref[...]
    for b in range(nb):
        w_b = w_ref[b, :][None, :]
        xb = x * w_b
        n = jnp.sqrt(jnp.sum(xb * xb, axis=1, keepdims=True))
        o_ref[:, b * d:(b + 1) * d] = (xb * w_b) / jnp.maximum(n, 1e-12)


def _zn(x, w):
    R, D = x.shape
    NB = w.shape[0]
    return pl.pallas_call(
        functools.partial(_zn_body, NB, D),
        in_specs=[pl.BlockSpec((R, D), lambda: (0, 0)),
                  pl.BlockSpec((NB, D), lambda: (0, 0))],
        out_specs=pl.BlockSpec((R, NB * D), lambda: (0, 0)),
        out_shape=jax.ShapeDtypeStruct((R, NB * D), jnp.float32),
    )(x, w)


def _att_body(nb, d, na, lowp, x_ref, w_ref, za_ref, old_ref, a_ref, b_ref,
              o_ref):
    x = x_ref[...]
    w = w_ref[...]
    nrm2 = lax.dot_general(x * x, w * w, (((1,), (1,)), ((), ())),
                           preferred_element_type=jnp.float32)
    rinv = 1.0 / jnp.maximum(jnp.sqrt(nrm2), 1e-12)
    xd = x.astype(jnp.bfloat16) if lowp else x
    acc = jnp.zeros((x.shape[0], za_ref.shape[0]), jnp.float32)
    for b in range(nb):
        za_b = za_ref[:, b * d:(b + 1) * d]
        if lowp:
            za_b = za_b.astype(jnp.bfloat16)
        g = lax.dot_general(xd, za_b, (((1,), (1,)), ((), ())),
                            preferred_element_type=jnp.float32)
        acc = acc + g * rinv[:, b:b + 1]
    att = acc * (1.0 / nb)
    col = lax.broadcasted_iota(jnp.int32, att.shape, 1)
    new = jnp.where((att > EPS) & (col < na), att, 0.0)
    o_ref[...] = a_ref[...] * old_ref[...] + b_ref[...] * new


def _knn_blend(x, w, za, old, alpha, beta, n_anchor, lowp):
    R, D = x.shape
    NB = w.shape[0]
    AP = za.shape[0]
    bm = 512
    return pl.pallas_call(
        functools.partial(_att_body, NB, D, n_anchor, lowp),
        grid=(R // bm,),
        in_specs=[pl.BlockSpec((bm, D), lambda i: (i, 0)),
                  pl.BlockSpec((NB, D), lambda i: (0, 0)),
                  pl.BlockSpec((AP, NB * D), lambda i: (0, 0)),
                  pl.BlockSpec((bm, AP), lambda i: (i, 0)),
                  pl.BlockSpec((bm, 1), lambda i: (i, 0)),
                  pl.BlockSpec((bm, 1), lambda i: (i, 0))],
        out_specs=pl.BlockSpec((bm, AP), lambda i: (i, 0)),
        out_shape=jax.ShapeDtypeStruct((R, AP), jnp.float32),
    )(x, w, za, old, alpha, beta)


def _atb_body(naa_ref, fw_ref, o_ref):
    o_ref[...] = lax.dot_general(naa_ref[...].astype(jnp.bfloat16),
                                 fw_ref[...].astype(jnp.bfloat16),
                                 (((0,), (0,)), ((), ())),
                                 preferred_element_type=jnp.float32)


def _atb(naa, fw):
    N, AP = naa.shape
    O = fw.shape[1]
    return pl.pallas_call(
        _atb_body,
        in_specs=[pl.BlockSpec((N, AP), lambda: (0, 0)),
                  pl.BlockSpec((N, O), lambda: (0, 0))],
        out_specs=pl.BlockSpec((AP, O), lambda: (0, 0)),
        out_shape=jax.ShapeDtypeStruct((AP, O), jnp.float32),
    )(naa, fw)


def _fuse_body(naa_ref, bm_ref, nvd_ref, fr_ref, o_ref):
    av = jax.nn.relu(jnp.dot(naa_ref[...].astype(jnp.bfloat16),
                             bm_ref[...].astype(jnp.bfloat16),
                             preferred_element_type=jnp.float32))
    fr = fr_ref[0, 0]
    o_ref[...] = fr * av + (1.0 - fr) * nvd_ref[...]


def _fuse(naa, bmat, nvd, fr_arr):
    N, AP = naa.shape
    O = bmat.shape[1]
    bm = 512
    return pl.pallas_call(
        _fuse_body,
        grid=(N // bm,),
        in_specs=[pl.BlockSpec((bm, AP), lambda i: (i, 0)),
                  pl.BlockSpec((AP, O), lambda i: (0, 0)),
                  pl.BlockSpec((bm, O), lambda i: (i, 0)),
                  pl.BlockSpec((1, 1), lambda i: (0, 0))],
        out_specs=pl.BlockSpec((bm, O), lambda i: (i, 0)),
        out_shape=jax.ShapeDtypeStruct((N, O), jnp.float32),
    )(naa, bmat, nvd, fr_arr)


def _sc_gather(table, idx):
    N, D = table.shape
    B = idx.shape[0]
    info = plsc.get_sparse_core_info()
    nc, ns = info.num_cores, info.num_subcores
    nw = nc * ns
    bpw = B // nw
    mesh = plsc.VectorSubcoreMesh(core_axis_name="c", subcore_axis_name="s")

    @functools.partial(
        pl.kernel, mesh=mesh,
        out_type=jax.ShapeDtypeStruct((B, D), jnp.float32),
        scratch_types=[
            pltpu.VMEM((bpw,), jnp.int32),
            pltpu.VMEM((bpw, D), jnp.float32),
            pltpu.SemaphoreType.DMA,
        ],
    )
    def gather_k(table_hbm, idx_hbm, out_hbm, idx_v, rows_v, sem):
        wid = lax.axis_index("s") * nc + lax.axis_index("c")
        base = wid * bpw
        pltpu.sync_copy(idx_hbm.at[pl.ds(base, bpw)], idx_v)
        pltpu.async_copy(table_hbm.at[idx_v], rows_v, sem).wait()
        pltpu.sync_copy(rows_v, out_hbm.at[pl.ds(base, bpw)])

    return gather_k(table, idx)


def kernel(features, ori_adj, anchor_nodes_idx, fusion_ratio, W1, W2,
           W_score, W_enc):
    N, D = features.shape
    O = W_enc.shape[1]
    A = anchor_nodes_idx.shape[0]
    KS = [0.9, 0.7]
    k1 = max(2, int(KS[0] * N))
    k2 = max(2, int(KS[1] * k1))
    AP = 512

    wcat = jnp.concatenate([W_enc, W_score], axis=1)
    fws = _proj(features, wcat)
    fw = fws[:, :O]
    s_col = fws[:, O:O + 1]
    nvd, y1_col = _big(ori_adj, fw, s_col)
    y1 = y1_col[:, 0]

    v1 = jax.nn.sigmoid(y1).reshape(N // 128, 128)
    ones = jnp.ones((N // 128, 128), jnp.float32)
    m1 = _topk_mask(v1, ones, k1)
    m1_col = m1.reshape(N, 1)

    y2 = _masked_matvec(ori_adj, s_col, m1_col)
    v2 = jax.nn.sigmoid(y2[:, 0]).reshape(N // 128, 128)
    m2 = _topk_mask(v2, m1, k2)
    m2_col = m2.reshape(N, 1)

    idx_pad = jnp.concatenate(
        [anchor_nodes_idx.astype(jnp.int32),
         jnp.zeros((AP - A,), jnp.int32)])

    xa = _sc_gather(features, idx_pad)
    za1 = _zn(xa, W1)
    zeros_naa = jnp.zeros((N, AP), jnp.float32)
    zero_col = jnp.zeros((N, 1), jnp.float32)
    one_col = jnp.ones((N, 1), jnp.float32)
    naa = _knn_blend(features, W1, za1, zeros_naa, zero_col, one_col, A,
                     lowp=False)

    fr_arr = jnp.asarray(fusion_ratio, jnp.float32).reshape(1, 1)
    for m_col in (m2_col, m1_col):
        bmat = _atb(naa, fw)
        nv = _fuse(naa, bmat, nvd, fr_arr)
        nva = _sc_gather(nv, idx_pad)
        za2 = _zn(nva, W2)
        alpha = 1.0 - 0.05 * m_col
        beta = 0.05 * m_col
        naa = _knn_blend(nv, W2, za2, naa, alpha, beta, A, lowp=True)

    return naa[:, :A]


# speedup vs baseline: 1.2577x; 1.2577x over previous
import functools

import jax
import jax.numpy as jnp
from jax import lax
from jax.experimental import pallas as pl
from jax.experimental.pallas import tpu as pltpu
from jax.experimental.pallas import tpu_sc as plsc

EPS = 0.1


def _prep_body(adj_ref, feat_ref, wcat_ref, nvd_ref, y1_ref, fws_ref, fws_s):
    i = pl.program_id(0)

    @pl.when(i == 0)
    def _():
        fws_s[...] = jnp.dot(feat_ref[...], wcat_ref[...],
                             preferred_element_type=jnp.float32)
        fws_ref[...] = fws_s[...]

    @pl.when(i > 0)
    def _():
        adj = adj_ref[...]
        p = jnp.dot(adj.astype(jnp.bfloat16),
                    fws_s[:, :256].astype(jnp.bfloat16),
                    preferred_element_type=jnp.float32)
        nvd_ref[...] = jax.nn.relu(p)
        y1_ref[...] = jnp.dot(adj, fws_s[:, 256:257],
                              preferred_element_type=jnp.float32)


def _prep(ori_adj, features, wcat):
    N, D = features.shape
    F = wcat.shape[1]
    bm = 512
    return pl.pallas_call(
        _prep_body,
        grid=(N // bm + 1,),
        in_specs=[pl.BlockSpec((bm, N), lambda i: (jnp.maximum(i - 1, 0), 0)),
                  pl.BlockSpec((N, D), lambda i: (0, 0)),
                  pl.BlockSpec((D, F), lambda i: (0, 0))],
        out_specs=[pl.BlockSpec((bm, 256),
                                lambda i: (jnp.maximum(i - 1, 0), 0)),
                   pl.BlockSpec((bm, 1), lambda i: (jnp.maximum(i - 1, 0), 0)),
                   pl.BlockSpec((N, F), lambda i: (0, 0))],
        out_shape=[jax.ShapeDtypeStruct((N, 256), jnp.float32),
                   jax.ShapeDtypeStruct((N, 1), jnp.float32),
                   jax.ShapeDtypeStruct((N, F), jnp.float32)],
        scratch_shapes=[pltpu.VMEM((N, F), jnp.float32)],
    )(ori_adj, features, wcat)


def _mv_body(adj_ref, s_ref, m_ref, o_ref):
    sm = s_ref[...] * m_ref[...]
    o_ref[...] = jnp.dot(adj_ref[...], sm, preferred_element_type=jnp.float32)


def _masked_matvec(ori_adj, s_col, m_col):
    N = ori_adj.shape[0]
    bm = 512
    return pl.pallas_call(
        _mv_body,
        grid=(N // bm,),
        in_specs=[pl.BlockSpec((bm, N), lambda i: (i, 0)),
                  pl.BlockSpec((N, 1), lambda i: (0, 0)),
                  pl.BlockSpec((N, 1), lambda i: (0, 0))],
        out_specs=pl.BlockSpec((bm, 1), lambda i: (i, 0)),
        out_shape=jax.ShapeDtypeStruct((N, 1), jnp.float32),
    )(ori_adj, s_col, m_col)


def _greedy_max(key, k, steps):
    prefix = jnp.int32(0)
    for shift, tmax in steps:
        rows = []
        for t in range(1, tmax + 1):
            cand = prefix | jnp.int32(t << shift)
            rows.append(jnp.sum((key >= cand).astype(jnp.int32), axis=0,
                                keepdims=True))
        counts = jnp.sum(jnp.concatenate(rows, axis=0), axis=1)
        nib = jnp.sum((counts >= k).astype(jnp.int32))
        prefix = prefix | lax.shift_left(nib, jnp.int32(shift))
    return prefix


_VAL_STEPS = ((28, 3), (24, 15), (20, 15), (16, 15), (12, 15), (8, 15),
              (4, 15), (0, 15))
_IDX_STEPS = ((12, 1), (8, 15), (4, 15), (0, 15))


def _select_body(k, v_ref, prev_ref, o_ref):
    v = v_ref[...]
    bits = lax.bitcast_convert_type(v, jnp.int32)
    key = jnp.where(prev_ref[...] > 0.5, bits, jnp.int32(-1))
    kk = jnp.int32(k)
    kth = _greedy_max(key, kk, _VAL_STEPS)
    chigh = jnp.sum((key > kth).astype(jnp.int32))
    r = kk - chigh
    tie = key == kth
    rr = lax.broadcasted_iota(jnp.int32, v.shape, 0)
    cc = lax.broadcasted_iota(jnp.int32, v.shape, 1)
    nel = v.shape[0] * v.shape[1]
    key2 = jnp.where(tie, jnp.int32(nel) - (rr * v.shape[1] + cc),
                     jnp.int32(0))
    kth2 = _greedy_max(key2, r, _IDX_STEPS)
    sel = (key > kth) | (key2 >= kth2)
    o_ref[...] = sel.astype(jnp.float32)


def _topk_mask(v, prev, k):
    R, C = v.shape
    return pl.pallas_call(
        functools.partial(_select_body, k),
        in_specs=[pl.BlockSpec((R, C), lambda: (0, 0)),
                  pl.BlockSpec((R, C), lambda: (0, 0))],
        out_specs=pl.BlockSpec((R, C), lambda: (0, 0)),
        out_shape=jax.ShapeDtypeStruct((R, C), jnp.float32),
    )(v, prev)


def _sc_gather(table, idx):
    N, D = table.shape
    B = idx.shape[0]
    info = plsc.get_sparse_core_info()
    nc, ns = info.num_cores, info.num_subcores
    nw = nc * ns
    bpw = B // nw
    mesh = plsc.VectorSubcoreMesh(core_axis_name="c", subcore_axis_name="s")

    @functools.partial(
        pl.kernel, mesh=mesh,
        out_type=jax.ShapeDtypeStruct((B, D), jnp.float32),
        scratch_types=[
            pltpu.VMEM((bpw,), jnp.int32),
            pltpu.VMEM((bpw, D), jnp.float32),
            pltpu.SemaphoreType.DMA,
        ],
    )
    def gather_k(table_hbm, idx_hbm, out_hbm, idx_v, rows_v, sem):
        wid = lax.axis_index("s") * nc + lax.axis_index("c")
        base = wid * bpw
        pltpu.sync_copy(idx_hbm.at[pl.ds(base, bpw)], idx_v)
        pltpu.async_copy(table_hbm.at[idx_v], rows_v, sem).wait()
        pltpu.sync_copy(rows_v, out_hbm.at[pl.ds(base, bpw)])

    return gather_k(table, idx)


def _zn_block(x, w_b):
    xb = x * w_b
    n = jnp.sqrt(jnp.sum(xb * xb, axis=1, keepdims=True))
    return (xb * w_b) / jnp.maximum(n, 1e-12)


def _knn1_body(nb, d, na, x_ref, w_ref, xa_ref, o_ref, za_s):
    i = pl.program_id(0)

    @pl.when(i == 0)
    def _():
        xa = xa_ref[...]
        for b in range(nb):
            za_s[:, b * d:(b + 1) * d] = _zn_block(xa, w_ref[b, :][None, :])

    @pl.when(i > 0)
    def _():
        x = x_ref[...]
        w = w_ref[...]
        nrm2 = lax.dot_general(x * x, w * w, (((1,), (1,)), ((), ())),
                               preferred_element_type=jnp.float32)
        rinv = 1.0 / jnp.maximum(jnp.sqrt(nrm2), 1e-12)
        acc = jnp.zeros((x.shape[0], za_s.shape[0]), jnp.float32)
        for b in range(nb):
            g = lax.dot_general(x, za_s[:, b * d:(b + 1) * d],
                                (((1,), (1,)), ((), ())),
                                preferred_element_type=jnp.float32)
            acc = acc + g * rinv[:, b:b + 1]
        att = acc * (1.0 / nb)
        col = lax.broadcasted_iota(jnp.int32, att.shape, 1)
        o_ref[...] = jnp.where((att > EPS) & (col < na), att, 0.0)


def _knn1(x, w, xa, n_anchor):
    R, D = x.shape
    NB = w.shape[0]
    AP = xa.shape[0]
    bm = 512
    return pl.pallas_call(
        functools.partial(_knn1_body, NB, D, n_anchor),
        grid=(R // bm + 1,),
        in_specs=[pl.BlockSpec((bm, D), lambda i: (jnp.maximum(i - 1, 0), 0)),
                  pl.BlockSpec((NB, D), lambda i: (0, 0)),
                  pl.BlockSpec((AP, D), lambda i: (0, 0))],
        out_specs=pl.BlockSpec((bm, AP), lambda i: (jnp.maximum(i - 1, 0), 0)),
        out_shape=jax.ShapeDtypeStruct((R, AP), jnp.float32),
        scratch_shapes=[pltpu.VMEM((AP, NB * D), jnp.float32)],
    )(x, w, xa)


def _refine_body(nb, d, na, nblk, bm,
                 naa_ref, fw_ref, nvd_ref, pb_ref, w_ref, m2_ref, m1_ref,
                 fr_ref, out_ref, bmat_s, nv_s, za_s, naa_s):
    fr = fr_ref[0, 0]
    w = w_ref[...]
    wsq = w * w
    col = lax.broadcasted_iota(jnp.int32, (bm, naa_ref.shape[1]), 1)
    for it, (src, dst, m_ref) in enumerate(
            [(naa_ref, naa_s, m2_ref), (naa_s, out_ref, m1_ref)]):
        bmat_s[...] = jnp.zeros_like(bmat_s)
        for blk in range(nblk):
            a = src[pl.ds(blk * bm, bm), :]
            f = fw_ref[pl.ds(blk * bm, bm), :]
            bmat_s[...] += lax.dot_general(
                a.astype(jnp.bfloat16), f.astype(jnp.bfloat16),
                (((0,), (0,)), ((), ())), preferred_element_type=jnp.float32)
        bmat = bmat_s[...].astype(jnp.bfloat16)
        for blk in range(nblk):
            a = src[pl.ds(blk * bm, bm), :]
            av = jax.nn.relu(jnp.dot(a.astype(jnp.bfloat16), bmat,
                                     preferred_element_type=jnp.float32))
            nv_s[pl.ds(blk * bm, bm), :] = (
                fr * av + (1.0 - fr) * nvd_ref[pl.ds(blk * bm, bm), :])
        nva = jnp.zeros((za_s.shape[0], d), jnp.float32)
        for blk in range(nblk):
            nva = nva + jnp.dot(
                pb_ref[:, pl.ds(blk * bm, bm)],
                nv_s[pl.ds(blk * bm, bm), :].astype(jnp.bfloat16),
                preferred_element_type=jnp.float32)
        for b in range(nb):
            za_s[:, b * d:(b + 1) * d] = _zn_block(nva, w[b, :][None, :])
        for blk in range(nblk):
            x = nv_s[pl.ds(blk * bm, bm), :]
            nrm2 = lax.dot_general(x * x, wsq, (((1,), (1,)), ((), ())),
                                   preferred_element_type=jnp.float32)
            rinv = 1.0 / jnp.maximum(jnp.sqrt(nrm2), 1e-12)
            xd = x.astype(jnp.bfloat16)
            acc = jnp.zeros((bm, za_s.shape[0]), jnp.float32)
            for b in range(nb):
                g = lax.dot_general(
                    xd, za_s[:, b * d:(b + 1) * d].astype(jnp.bfloat16),
                    (((1,), (1,)), ((), ())),
                    preferred_element_type=jnp.float32)
                acc = acc + g * rinv[:, b:b + 1]
            att = acc * (1.0 / nb)
            new = jnp.where((att > EPS) & (col < na), att, 0.0)
            m = m_ref[pl.ds(blk * bm, bm), :]
            old = src[pl.ds(blk * bm, bm), :]
            dst[pl.ds(blk * bm, bm), :] = (
                (1.0 - 0.05 * m) * old + (0.05 * m) * new)


def _refine(naa, fw, nvd, pb, w, m2_col, m1_col, fr_arr, n_anchor):
    N, AP = naa.shape
    O = fw.shape[1]
    NB = w.shape[0]
    bm = 512
    return pl.pallas_call(
        functools.partial(_refine_body, NB, O, n_anchor, N // bm, bm),
        in_specs=[pl.BlockSpec((N, AP), lambda: (0, 0)),
                  pl.BlockSpec((N, O), lambda: (0, 0)),
                  pl.BlockSpec((N, O), lambda: (0, 0)),
                  pl.BlockSpec((AP, N), lambda: (0, 0)),
                  pl.BlockSpec((NB, O), lambda: (0, 0)),
                  pl.BlockSpec((N, 1), lambda: (0, 0)),
                  pl.BlockSpec((N, 1), lambda: (0, 0)),
                  pl.BlockSpec((1, 1), lambda: (0, 0))],
        out_specs=pl.BlockSpec((N, AP), lambda: (0, 0)),
        out_shape=jax.ShapeDtypeStruct((N, AP), jnp.float32),
        scratch_shapes=[pltpu.VMEM((AP, O), jnp.float32),
                        pltpu.VMEM((N, O), jnp.float32),
                        pltpu.VMEM((AP, NB * O), jnp.float32),
                        pltpu.VMEM((N, AP), jnp.float32)],
    )(naa, fw, nvd, pb, w, m2_col, m1_col, fr_arr)


def kernel(features, ori_adj, anchor_nodes_idx, fusion_ratio, W1, W2,
           W_score, W_enc):
    N, D = features.shape
    O = W_enc.shape[1]
    A = anchor_nodes_idx.shape[0]
    KS = [0.9, 0.7]
    k1 = max(2, int(KS[0] * N))
    k2 = max(2, int(KS[1] * k1))
    AP = 512

    wcat = jnp.concatenate(
        [W_enc, W_score, jnp.zeros((D, 384 - O - 1), jnp.float32)], axis=1)
    nvd, y1_col, fws = _prep(ori_adj, features, wcat)
    fw = fws[:, :O]
    s_col = fws[:, O:O + 1]

    v1 = jax.nn.sigmoid(y1_col[:, 0]).reshape(N // 128, 128)
    ones = jnp.ones((N // 128, 128), jnp.float32)
    m1 = _topk_mask(v1, ones, k1)
    m1_col = m1.reshape(N, 1)

    y2 = _masked_matvec(ori_adj, s_col, m1_col)
    v2 = jax.nn.sigmoid(y2[:, 0]).reshape(N // 128, 128)
    m2 = _topk_mask(v2, m1, k2)
    m2_col = m2.reshape(N, 1)

    idx_pad = jnp.concatenate(
        [anchor_nodes_idx.astype(jnp.int32), jnp.zeros((AP - A,), jnp.int32)])

    xa = _sc_gather(features, idx_pad)
    naa = _knn1(features, W1, xa, A)

    pb = jax.nn.one_hot(idx_pad, N, dtype=jnp.bfloat16)
    fr_arr = jnp.asarray(fusion_ratio, jnp.float32).reshape(1, 1)
    out = _refine(naa, fw, nvd, pb, W2, m2_col, m1_col, fr_arr, A)
    return out[:, :A]


# speedup vs baseline: 1.2587x; 1.0008x over previous
import functools

import jax
import jax.numpy as jnp
from jax import lax
from jax.experimental import pallas as pl
from jax.experimental.pallas import tpu as pltpu
from jax.experimental.pallas import tpu_sc as plsc

EPS = 0.1


def _prep_body(adj_ref, feat_ref, wcat_ref, nvd_ref, y1_ref, fws_ref, fws_s):
    i = pl.program_id(0)

    @pl.when(i == 0)
    def _():
        fws_s[...] = jnp.dot(feat_ref[...], wcat_ref[...],
                             preferred_element_type=jnp.float32)
        fws_ref[...] = fws_s[...]

    @pl.when(i > 0)
    def _():
        adj = adj_ref[...]
        p = jnp.dot(adj.astype(jnp.bfloat16),
                    fws_s[:, :256].astype(jnp.bfloat16),
                    preferred_element_type=jnp.float32)
        nvd_ref[...] = jax.nn.relu(p)
        y1_ref[...] = jnp.dot(adj, fws_s[:, 256:257],
                              preferred_element_type=jnp.float32)


def _prep(ori_adj, features, wcat):
    N, D = features.shape
    F = wcat.shape[1]
    bm = 512
    return pl.pallas_call(
        _prep_body,
        grid=(N // bm + 1,),
        in_specs=[pl.BlockSpec((bm, N), lambda i: (jnp.maximum(i - 1, 0), 0)),
                  pl.BlockSpec((N, D), lambda i: (0, 0)),
                  pl.BlockSpec((D, F), lambda i: (0, 0))],
        out_specs=[pl.BlockSpec((bm, 256),
                                lambda i: (jnp.maximum(i - 1, 0), 0)),
                   pl.BlockSpec((bm, 1), lambda i: (jnp.maximum(i - 1, 0), 0)),
                   pl.BlockSpec((N, F), lambda i: (0, 0))],
        out_shape=[jax.ShapeDtypeStruct((N, 256), jnp.float32),
                   jax.ShapeDtypeStruct((N, 1), jnp.float32),
                   jax.ShapeDtypeStruct((N, F), jnp.float32)],
        scratch_shapes=[pltpu.VMEM((N, F), jnp.float32)],
    )(ori_adj, features, wcat)


def _mv_body(adj_ref, s_ref, m_ref, o_ref):
    sm = s_ref[...] * m_ref[...]
    o_ref[...] = jnp.dot(adj_ref[...], sm, preferred_element_type=jnp.float32)


def _masked_matvec(ori_adj, s_col, m_col):
    N = ori_adj.shape[0]
    bm = 512
    return pl.pallas_call(
        _mv_body,
        grid=(N // bm,),
        in_specs=[pl.BlockSpec((bm, N), lambda i: (i, 0)),
                  pl.BlockSpec((N, 1), lambda i: (0, 0)),
                  pl.BlockSpec((N, 1), lambda i: (0, 0))],
        out_specs=pl.BlockSpec((bm, 1), lambda i: (i, 0)),
        out_shape=jax.ShapeDtypeStruct((N, 1), jnp.float32),
    )(ori_adj, s_col, m_col)


def _greedy_max(key, k, steps):
    prefix = jnp.int32(0)
    for shift, tmax in steps:
        rows = []
        for t in range(1, tmax + 1):
            cand = prefix | jnp.int32(t << shift)
            rows.append(jnp.sum((key >= cand).astype(jnp.int32), axis=0,
                                keepdims=True))
        counts = jnp.sum(jnp.concatenate(rows, axis=0), axis=1)
        nib = jnp.sum((counts >= k).astype(jnp.int32))
        prefix = prefix | lax.shift_left(nib, jnp.int32(shift))
    return prefix


_VAL_STEPS = ((28, 3), (24, 15), (20, 15), (16, 15), (12, 15), (8, 15),
              (4, 15), (0, 15))
_IDX_STEPS = ((12, 1), (8, 15), (4, 15), (0, 15))


def _select_body(k, v_ref, prev_ref, o_ref):
    v = v_ref[...]
    bits = lax.bitcast_convert_type(v, jnp.int32)
    key = jnp.where(prev_ref[...] > 0.5, bits, jnp.int32(-1))
    kk = jnp.int32(k)
    kth = _greedy_max(key, kk, _VAL_STEPS)
    chigh = jnp.sum((key > kth).astype(jnp.int32))
    r = kk - chigh
    tie = key == kth
    rr = lax.broadcasted_iota(jnp.int32, v.shape, 0)
    cc = lax.broadcasted_iota(jnp.int32, v.shape, 1)
    nel = v.shape[0] * v.shape[1]
    key2 = jnp.where(tie, jnp.int32(nel) - (rr * v.shape[1] + cc),
                     jnp.int32(0))
    kth2 = _greedy_max(key2, r, _IDX_STEPS)
    sel = (key > kth) | (key2 >= kth2)
    o_ref[...] = sel.astype(jnp.float32)


def _topk_mask(v, prev, k):
    R, C = v.shape
    return pl.pallas_call(
        functools.partial(_select_body, k),
        in_specs=[pl.BlockSpec((R, C), lambda: (0, 0)),
                  pl.BlockSpec((R, C), lambda: (0, 0))],
        out_specs=pl.BlockSpec((R, C), lambda: (0, 0)),
        out_shape=jax.ShapeDtypeStruct((R, C), jnp.float32),
    )(v, prev)


def _sc_gather(table, idx):
    N, D = table.shape
    B = idx.shape[0]
    info = plsc.get_sparse_core_info()
    nc, ns = info.num_cores, info.num_subcores
    nw = nc * ns
    bpw = B // nw
    mesh = plsc.VectorSubcoreMesh(core_axis_name="c", subcore_axis_name="s")

    @functools.partial(
        pl.kernel, mesh=mesh,
        out_type=jax.ShapeDtypeStruct((B, D), jnp.float32),
        scratch_types=[
            pltpu.VMEM((bpw,), jnp.int32),
            pltpu.VMEM((bpw, D), jnp.float32),
            pltpu.SemaphoreType.DMA,
        ],
    )
    def gather_k(table_hbm, idx_hbm, out_hbm, idx_v, rows_v, sem):
        wid = lax.axis_index("s") * nc + lax.axis_index("c")
        base = wid * bpw
        pltpu.sync_copy(idx_hbm.at[pl.ds(base, bpw)], idx_v)
        pltpu.async_copy(table_hbm.at[idx_v], rows_v, sem).wait()
        pltpu.sync_copy(rows_v, out_hbm.at[pl.ds(base, bpw)])

    return gather_k(table, idx)


def _zn_block(x, w_b):
    xb = x * w_b
    n = jnp.sqrt(jnp.sum(xb * xb, axis=1, keepdims=True))
    return (xb * w_b) / jnp.maximum(n, 1e-12)


def _l2n_block(x, w_b):
    xb = x * w_b
    n = jnp.sqrt(jnp.sum(xb * xb, axis=1, keepdims=True))
    return xb / jnp.maximum(n, 1e-12)


def _knn1_body(nb, d, na, x_ref, w_ref, xa_ref, o_ref, za_s):
    i = pl.program_id(0)

    @pl.when(i == 0)
    def _():
        xa = xa_ref[...]
        for b in range(nb):
            za_s[:, b * d:(b + 1) * d] = _l2n_block(xa, w_ref[b, :][None, :])

    @pl.when(i > 0)
    def _():
        x = x_ref[...]
        acc = jnp.zeros((x.shape[0], za_s.shape[0]), jnp.float32)
        for b in range(nb):
            xn = _l2n_block(x, w_ref[b, :][None, :])
            acc = acc + lax.dot_general(xn, za_s[:, b * d:(b + 1) * d],
                                        (((1,), (1,)), ((), ())),
                                        preferred_element_type=jnp.float32)
        att = acc * (1.0 / nb)
        col = lax.broadcasted_iota(jnp.int32, att.shape, 1)
        o_ref[...] = jnp.where((att > EPS) & (col < na), att, 0.0)


def _knn1(x, w, xa, n_anchor):
    R, D = x.shape
    NB = w.shape[0]
    AP = xa.shape[0]
    bm = 512
    return pl.pallas_call(
        functools.partial(_knn1_body, NB, D, n_anchor),
        grid=(R // bm + 1,),
        in_specs=[pl.BlockSpec((bm, D), lambda i: (jnp.maximum(i - 1, 0), 0)),
                  pl.BlockSpec((NB, D), lambda i: (0, 0)),
                  pl.BlockSpec((AP, D), lambda i: (0, 0))],
        out_specs=pl.BlockSpec((bm, AP), lambda i: (jnp.maximum(i - 1, 0), 0)),
        out_shape=jax.ShapeDtypeStruct((R, AP), jnp.float32),
        scratch_shapes=[pltpu.VMEM((AP, NB * D), jnp.float32)],
    )(x, w, xa)


def _refine_body(nb, d, na, nblk, bm,
                 naa_ref, fw_ref, nvd_ref, pb_ref, w_ref, m2_ref, m1_ref,
                 fr_ref, out_ref, bmat_s, nv_s, za_s, naa_s):
    fr = fr_ref[0, 0]
    w = w_ref[...]
    wsq = w * w
    col = lax.broadcasted_iota(jnp.int32, (bm, naa_ref.shape[1]), 1)
    for it, (src, dst, m_ref) in enumerate(
            [(naa_ref, naa_s, m2_ref), (naa_s, out_ref, m1_ref)]):
        bmat_s[...] = jnp.zeros_like(bmat_s)
        for blk in range(nblk):
            a = src[pl.ds(blk * bm, bm), :]
            f = fw_ref[pl.ds(blk * bm, bm), :]
            bmat_s[...] += lax.dot_general(
                a.astype(jnp.bfloat16), f.astype(jnp.bfloat16),
                (((0,), (0,)), ((), ())), preferred_element_type=jnp.float32)
        bmat = bmat_s[...].astype(jnp.bfloat16)
        for blk in range(nblk):
            a = src[pl.ds(blk * bm, bm), :]
            av = jax.nn.relu(jnp.dot(a.astype(jnp.bfloat16), bmat,
                                     preferred_element_type=jnp.float32))
            nv_s[pl.ds(blk * bm, bm), :] = (
                fr * av + (1.0 - fr) * nvd_ref[pl.ds(blk * bm, bm), :])
        nva = jnp.zeros((za_s.shape[0], d), jnp.float32)
        for blk in range(nblk):
            nva = nva + jnp.dot(
                pb_ref[:, pl.ds(blk * bm, bm)],
                nv_s[pl.ds(blk * bm, bm), :].astype(jnp.bfloat16),
                preferred_element_type=jnp.float32)
        for b in range(nb):
            za_s[:, b * d:(b + 1) * d] = _zn_block(nva, w[b, :][None, :])
        for blk in range(nblk):
            x = nv_s[pl.ds(blk * bm, bm), :]
            nrm2 = lax.dot_general(x * x, wsq, (((1,), (1,)), ((), ())),
                                   preferred_element_type=jnp.float32)
            rinv = 1.0 / jnp.maximum(jnp.sqrt(nrm2), 1e-12)
            xd = x.astype(jnp.bfloat16)
            acc = jnp.zeros((bm, za_s.shape[0]), jnp.float32)
            for b in range(nb):
                g = lax.dot_general(
                    xd, za_s[:, b * d:(b + 1) * d].astype(jnp.bfloat16),
                    (((1,), (1,)), ((), ())),
                    preferred_element_type=jnp.float32)
                acc = acc + g * rinv[:, b:b + 1]
            att = acc * (1.0 / nb)
            new = jnp.where((att > EPS) & (col < na), att, 0.0)
            m = m_ref[pl.ds(blk * bm, bm), :]
            old = src[pl.ds(blk * bm, bm), :]
            dst[pl.ds(blk * bm, bm), :] = (
                (1.0 - 0.05 * m) * old + (0.05 * m) * new)


def _refine(naa, fw, nvd, pb, w, m2_col, m1_col, fr_arr, n_anchor):
    N, AP = naa.shape
    O = fw.shape[1]
    NB = w.shape[0]
    bm = 512
    return pl.pallas_call(
        functools.partial(_refine_body, NB, O, n_anchor, N // bm, bm),
        in_specs=[pl.BlockSpec((N, AP), lambda: (0, 0)),
                  pl.BlockSpec((N, O), lambda: (0, 0)),
                  pl.BlockSpec((N, O), lambda: (0, 0)),
                  pl.BlockSpec((AP, N), lambda: (0, 0)),
                  pl.BlockSpec((NB, O), lambda: (0, 0)),
                  pl.BlockSpec((N, 1), lambda: (0, 0)),
                  pl.BlockSpec((N, 1), lambda: (0, 0)),
                  pl.BlockSpec((1, 1), lambda: (0, 0))],
        out_specs=pl.BlockSpec((N, AP), lambda: (0, 0)),
        out_shape=jax.ShapeDtypeStruct((N, AP), jnp.float32),
        scratch_shapes=[pltpu.VMEM((AP, O), jnp.float32),
                        pltpu.VMEM((N, O), jnp.float32),
                        pltpu.VMEM((AP, NB * O), jnp.float32),
                        pltpu.VMEM((N, AP), jnp.float32)],
    )(naa, fw, nvd, pb, w, m2_col, m1_col, fr_arr)


def kernel(features, ori_adj, anchor_nodes_idx, fusion_ratio, W1, W2,
           W_score, W_enc):
    N, D = features.shape
    O = W_enc.shape[1]
    A = anchor_nodes_idx.shape[0]
    KS = [0.9, 0.7]
    k1 = max(2, int(KS[0] * N))
    k2 = max(2, int(KS[1] * k1))
    AP = 512

    wcat = jnp.concatenate(
        [W_enc, W_score, jnp.zeros((D, 384 - O - 1), jnp.float32)], axis=1)
    nvd, y1_col, fws = _prep(ori_adj, features, wcat)
    fw = fws[:, :O]
    s_col = fws[:, O:O + 1]

    v1 = jax.nn.sigmoid(y1_col[:, 0]).reshape(N // 128, 128)
    ones = jnp.ones((N // 128, 128), jnp.float32)
    m1 = _topk_mask(v1, ones, k1)
    m1_col = m1.reshape(N, 1)

    y2 = _masked_matvec(ori_adj, s_col, m1_col)
    v2 = jax.nn.sigmoid(y2[:, 0]).reshape(N // 128, 128)
    m2 = _topk_mask(v2, m1, k2)
    m2_col = m2.reshape(N, 1)

    idx_pad = jnp.concatenate(
        [anchor_nodes_idx.astype(jnp.int32), jnp.zeros((AP - A,), jnp.int32)])

    xa = _sc_gather(features, idx_pad)
    naa = _knn1(features, W1, xa, A)

    pb = jax.nn.one_hot(idx_pad, N, dtype=jnp.bfloat16)
    fr_arr = jnp.asarray(fusion_ratio, jnp.float32).reshape(1, 1)
    out = _refine(naa, fw, nvd, pb, W2, m2_col, m1_col, fr_arr, A)
    return out[:, :A]


# speedup vs baseline: 1.2821x; 1.0186x over previous
import functools

import jax
import jax.numpy as jnp
from jax import lax
from jax.experimental import pallas as pl
from jax.experimental.pallas import tpu as pltpu
from jax.experimental.pallas import tpu_sc as plsc

EPS = 0.1


def _prep_body(adj_ref, feat_ref, wcat_ref, nvd_ref, y1_ref, fws_ref, fws_s):
    i = pl.program_id(0)

    @pl.when(i == 0)
    def _():
        fws_s[...] = jnp.dot(feat_ref[...], wcat_ref[...],
                             preferred_element_type=jnp.float32)
        fws_ref[...] = fws_s[...]

    @pl.when(i > 0)
    def _():
        adj = adj_ref[...]
        p = jnp.dot(adj.astype(jnp.bfloat16),
                    fws_s[:, :256].astype(jnp.bfloat16),
                    preferred_element_type=jnp.float32)
        nvd_ref[...] = jax.nn.relu(p)
        y1_ref[...] = jnp.dot(adj, fws_s[:, 256:257],
                              preferred_element_type=jnp.float32)


def _prep(ori_adj, features, wcat):
    N, D = features.shape
    F = wcat.shape[1]
    bm = 512
    return pl.pallas_call(
        _prep_body,
        grid=(N // bm + 1,),
        in_specs=[pl.BlockSpec((bm, N), lambda i: (jnp.maximum(i - 1, 0), 0)),
                  pl.BlockSpec((N, D), lambda i: (0, 0)),
                  pl.BlockSpec((D, F), lambda i: (0, 0))],
        out_specs=[pl.BlockSpec((bm, 256),
                                lambda i: (jnp.maximum(i - 1, 0), 0)),
                   pl.BlockSpec((bm, 1), lambda i: (jnp.maximum(i - 1, 0), 0)),
                   pl.BlockSpec((N, F), lambda i: (0, 0))],
        out_shape=[jax.ShapeDtypeStruct((N, 256), jnp.float32),
                   jax.ShapeDtypeStruct((N, 1), jnp.float32),
                   jax.ShapeDtypeStruct((N, F), jnp.float32)],
        scratch_shapes=[pltpu.VMEM((N, F), jnp.float32)],
    )(ori_adj, features, wcat)


def _selmv_body(k, adj_ref, v_ref, vc_ref, s_ref, m_ref, mc_ref, y2_ref,
                smc_s):
    i = pl.program_id(0)

    @pl.when(i == 0)
    def _():
        v = v_ref[...]
        nel = v.shape[0] * v.shape[1]
        kth, kth2 = _sel_thresholds(v, jnp.ones_like(v), k, nel)
        m_ref[...] = _sel_eval(v, jnp.ones_like(v), _idx32(v.shape),
                               kth, kth2, nel)
        vc = vc_ref[...]
        idxc = lax.broadcasted_iota(jnp.int32, vc.shape, 0)
        mc = _sel_eval(vc, jnp.ones_like(vc), idxc, kth, kth2, nel)
        mc_ref[...] = mc
        smc_s[...] = s_ref[...] * mc

    @pl.when(i > 0)
    def _():
        y2_ref[...] = jnp.dot(adj_ref[...], smc_s[...],
                              preferred_element_type=jnp.float32)


def _sel_matvec(ori_adj, v1, v1_col, s_col, k):
    N = ori_adj.shape[0]
    R, C = v1.shape
    bm = 512
    return pl.pallas_call(
        functools.partial(_selmv_body, k),
        grid=(N // bm + 1,),
        in_specs=[pl.BlockSpec((bm, N), lambda i: (jnp.maximum(i - 1, 0), 0)),
                  pl.BlockSpec((R, C), lambda i: (0, 0)),
                  pl.BlockSpec((N, 1), lambda i: (0, 0)),
                  pl.BlockSpec((N, 1), lambda i: (0, 0))],
        out_specs=[pl.BlockSpec((R, C), lambda i: (0, 0)),
                   pl.BlockSpec((N, 1), lambda i: (0, 0)),
                   pl.BlockSpec((bm, 1), lambda i: (jnp.maximum(i - 1, 0), 0))],
        out_shape=[jax.ShapeDtypeStruct((R, C), jnp.float32),
                   jax.ShapeDtypeStruct((N, 1), jnp.float32),
                   jax.ShapeDtypeStruct((N, 1), jnp.float32)],
        scratch_shapes=[pltpu.VMEM((N, 1), jnp.float32)],
    )(ori_adj, v1, v1_col, s_col)


def _greedy_max(key, k, steps):
    prefix = jnp.int32(0)
    for shift, tmax in steps:
        rows = []
        for t in range(1, tmax + 1):
            cand = prefix | jnp.int32(t << shift)
            rows.append(jnp.sum((key >= cand).astype(jnp.int32), axis=0,
                                keepdims=True))
        counts = jnp.sum(jnp.concatenate(rows, axis=0), axis=1)
        nib = jnp.sum((counts >= k).astype(jnp.int32))
        prefix = prefix | lax.shift_left(nib, jnp.int32(shift))
    return prefix


_VAL_STEPS = ((28, 3), (24, 15), (20, 15), (16, 15), (12, 15), (8, 15),
              (4, 15), (0, 15))
_IDX_STEPS = ((12, 1), (8, 15), (4, 15), (0, 15))


def _sel_thresholds(v, prev, k, nel):
    bits = lax.bitcast_convert_type(v, jnp.int32)
    key = jnp.where(prev > 0.5, bits, jnp.int32(-1))
    kk = jnp.int32(k)
    kth = _greedy_max(key, kk, _VAL_STEPS)
    chigh = jnp.sum((key > kth).astype(jnp.int32))
    r = kk - chigh
    tie = key == kth
    rr = lax.broadcasted_iota(jnp.int32, v.shape, 0)
    cc = lax.broadcasted_iota(jnp.int32, v.shape, 1)
    key2 = jnp.where(tie, jnp.int32(nel) - (rr * v.shape[1] + cc),
                     jnp.int32(0))
    kth2 = _greedy_max(key2, r, _IDX_STEPS)
    return kth, kth2


def _sel_eval(v, prev, idx, kth, kth2, nel):
    bits = lax.bitcast_convert_type(v, jnp.int32)
    key = jnp.where(prev > 0.5, bits, jnp.int32(-1))
    key2 = jnp.where(key == kth, jnp.int32(nel) - idx, jnp.int32(0))
    return ((key > kth) | (key2 >= kth2)).astype(jnp.float32)


def _idx32(shape):
    rr = lax.broadcasted_iota(jnp.int32, shape, 0)
    cc = lax.broadcasted_iota(jnp.int32, shape, 1)
    return rr * shape[1] + cc


def _select_body(k, v_ref, prev_ref, vc_ref, pc_ref, oc_ref):
    v = v_ref[...]
    nel = v.shape[0] * v.shape[1]
    kth, kth2 = _sel_thresholds(v, prev_ref[...], k, nel)
    vc = vc_ref[...]
    idxc = lax.broadcasted_iota(jnp.int32, vc.shape, 0)
    oc_ref[...] = _sel_eval(vc, pc_ref[...], idxc, kth, kth2, nel)


def _topk_mask_col(v, prev, v_col, prev_col, k):
    R, C = v.shape
    return pl.pallas_call(
        functools.partial(_select_body, k),
        in_specs=[pl.BlockSpec((R, C), lambda: (0, 0)),
                  pl.BlockSpec((R, C), lambda: (0, 0)),
                  pl.BlockSpec((R * C, 1), lambda: (0, 0)),
                  pl.BlockSpec((R * C, 1), lambda: (0, 0))],
        out_specs=pl.BlockSpec((R * C, 1), lambda: (0, 0)),
        out_shape=jax.ShapeDtypeStruct((R * C, 1), jnp.float32),
    )(v, prev, v_col, prev_col)


def _sc_gather(table, idx):
    N, D = table.shape
    B = idx.shape[0]
    info = plsc.get_sparse_core_info()
    nc, ns = info.num_cores, info.num_subcores
    nw = nc * ns
    bpw = B // nw
    mesh = plsc.VectorSubcoreMesh(core_axis_name="c", subcore_axis_name="s")

    @functools.partial(
        pl.kernel, mesh=mesh,
        out_type=jax.ShapeDtypeStruct((B, D), jnp.float32),
        scratch_types=[
            pltpu.VMEM((bpw,), jnp.int32),
            pltpu.VMEM((bpw, D), jnp.float32),
            pltpu.SemaphoreType.DMA,
        ],
    )
    def gather_k(table_hbm, idx_hbm, out_hbm, idx_v, rows_v, sem):
        wid = lax.axis_index("s") * nc + lax.axis_index("c")
        base = wid * bpw
        pltpu.sync_copy(idx_hbm.at[pl.ds(base, bpw)], idx_v)
        pltpu.async_copy(table_hbm.at[idx_v], rows_v, sem).wait()
        pltpu.sync_copy(rows_v, out_hbm.at[pl.ds(base, bpw)])

    return gather_k(table, idx)


def _zn_block(x, w_b):
    xb = x * w_b
    n = jnp.sqrt(jnp.sum(xb * xb, axis=1, keepdims=True))
    return (xb * w_b) / jnp.maximum(n, 1e-12)


def _l2n_block(x, w_b):
    xb = x * w_b
    n = jnp.sqrt(jnp.sum(xb * xb, axis=1, keepdims=True))
    return xb / jnp.maximum(n, 1e-12)


def _knn1_body(nb, d, na, x_ref, w_ref, xa_ref, o_ref, za_s):
    i = pl.program_id(0)

    @pl.when(i == 0)
    def _():
        xa = xa_ref[...]
        for b in range(nb):
            za_s[:, b * d:(b + 1) * d] = _l2n_block(xa, w_ref[b, :][None, :])

    @pl.when(i > 0)
    def _():
        x = x_ref[...]
        acc = jnp.zeros((x.shape[0], za_s.shape[0]), jnp.float32)
        for b in range(nb):
            xn = _l2n_block(x, w_ref[b, :][None, :])
            acc = acc + lax.dot_general(xn, za_s[:, b * d:(b + 1) * d],
                                        (((1,), (1,)), ((), ())),
                                        preferred_element_type=jnp.float32)
        att = acc * (1.0 / nb)
        col = lax.broadcasted_iota(jnp.int32, att.shape, 1)
        o_ref[...] = jnp.where((att > EPS) & (col < na), att, 0.0)


def _knn1(x, w, xa, n_anchor):
    R, D = x.shape
    NB = w.shape[0]
    AP = xa.shape[0]
    bm = 512
    return pl.pallas_call(
        functools.partial(_knn1_body, NB, D, n_anchor),
        grid=(R // bm + 1,),
        in_specs=[pl.BlockSpec((bm, D), lambda i: (jnp.maximum(i - 1, 0), 0)),
                  pl.BlockSpec((NB, D), lambda i: (0, 0)),
                  pl.BlockSpec((AP, D), lambda i: (0, 0))],
        out_specs=pl.BlockSpec((bm, AP), lambda i: (jnp.maximum(i - 1, 0), 0)),
        out_shape=jax.ShapeDtypeStruct((R, AP), jnp.float32),
        scratch_shapes=[pltpu.VMEM((AP, NB * D), jnp.float32)],
    )(x, w, xa)


def _refine_body(nb, d, na, nblk, bm,
                 naa_ref, fw_ref, nvd_ref, pb_ref, w_ref, m2_ref, m1_ref,
                 fr_ref, out_ref, bmat_s, nv_s, za_s, naa_s):
    fr = fr_ref[0, 0]
    w = w_ref[...]
    wsq = w * w
    col = lax.broadcasted_iota(jnp.int32, (bm, naa_ref.shape[1]), 1)
    for it, (src, dst, m_ref) in enumerate(
            [(naa_ref, naa_s, m2_ref), (naa_s, out_ref, m1_ref)]):
        bmat_s[...] = jnp.zeros_like(bmat_s)
        for blk in range(nblk):
            a = src[pl.ds(blk * bm, bm), :]
            f = fw_ref[pl.ds(blk * bm, bm), :]
            bmat_s[...] += lax.dot_general(
                a.astype(jnp.bfloat16), f.astype(jnp.bfloat16),
                (((0,), (0,)), ((), ())), preferred_element_type=jnp.float32)
        bmat = bmat_s[...].astype(jnp.bfloat16)
        for blk in range(nblk):
            a = src[pl.ds(blk * bm, bm), :]
            av = jax.nn.relu(jnp.dot(a.astype(jnp.bfloat16), bmat,
                                     preferred_element_type=jnp.float32))
            nv_s[pl.ds(blk * bm, bm), :] = (
                fr * av + (1.0 - fr) * nvd_ref[pl.ds(blk * bm, bm), :])
        nva = jnp.zeros((za_s.shape[0], d), jnp.float32)
        for blk in range(nblk):
            nva = nva + jnp.dot(
                pb_ref[:, pl.ds(blk * bm, bm)],
                nv_s[pl.ds(blk * bm, bm), :].astype(jnp.bfloat16),
                preferred_element_type=jnp.float32)
        for b in range(nb):
            za_s[:, b * d:(b + 1) * d] = _zn_block(nva, w[b, :][None, :])
        for blk in range(nblk):
            x = nv_s[pl.ds(blk * bm, bm), :]
            nrm2 = lax.dot_general(x * x, wsq, (((1,), (1,)), ((), ())),
                                   preferred_element_type=jnp.float32)
            rinv = 1.0 / jnp.maximum(jnp.sqrt(nrm2), 1e-12)
            xd = x.astype(jnp.bfloat16)
            acc = jnp.zeros((bm, za_s.shape[0]), jnp.float32)
            for b in range(nb):
                g = lax.dot_general(
                    xd, za_s[:, b * d:(b + 1) * d].astype(jnp.bfloat16),
                    (((1,), (1,)), ((), ())),
                    preferred_element_type=jnp.float32)
                acc = acc + g * rinv[:, b:b + 1]
            att = acc * (1.0 / nb)
            new = jnp.where((att > EPS) & (col < na), att, 0.0)
            m = m_ref[pl.ds(blk * bm, bm), :]
            old = src[pl.ds(blk * bm, bm), :]
            dst[pl.ds(blk * bm, bm), :] = (
                (1.0 - 0.05 * m) * old + (0.05 * m) * new)


def _refine(naa, fw, nvd, pb, w, m2_col, m1_col, fr_arr, n_anchor):
    N, AP = naa.shape
    O = fw.shape[1]
    NB = w.shape[0]
    bm = 512
    return pl.pallas_call(
        functools.partial(_refine_body, NB, O, n_anchor, N // bm, bm),
        in_specs=[pl.BlockSpec((N, AP), lambda: (0, 0)),
                  pl.BlockSpec((N, O), lambda: (0, 0)),
                  pl.BlockSpec((N, O), lambda: (0, 0)),
                  pl.BlockSpec((AP, N), lambda: (0, 0)),
                  pl.BlockSpec((NB, O), lambda: (0, 0)),
                  pl.BlockSpec((N, 1), lambda: (0, 0)),
                  pl.BlockSpec((N, 1), lambda: (0, 0)),
                  pl.BlockSpec((1, 1), lambda: (0, 0))],
        out_specs=pl.BlockSpec((N, AP), lambda: (0, 0)),
        out_shape=jax.ShapeDtypeStruct((N, AP), jnp.float32),
        scratch_shapes=[pltpu.VMEM((AP, O), jnp.float32),
                        pltpu.VMEM((N, O), jnp.float32),
                        pltpu.VMEM((AP, NB * O), jnp.float32),
                        pltpu.VMEM((N, AP), jnp.float32)],
    )(naa, fw, nvd, pb, w, m2_col, m1_col, fr_arr)


def kernel(features, ori_adj, anchor_nodes_idx, fusion_ratio, W1, W2,
           W_score, W_enc):
    N, D = features.shape
    O = W_enc.shape[1]
    A = anchor_nodes_idx.shape[0]
    KS = [0.9, 0.7]
    k1 = max(2, int(KS[0] * N))
    k2 = max(2, int(KS[1] * k1))
    AP = 512

    wcat = jnp.concatenate(
        [W_enc, W_score, jnp.zeros((D, 384 - O - 1), jnp.float32)], axis=1)
    nvd, y1_col, fws = _prep(ori_adj, features, wcat)
    fw = fws[:, :O]
    s_col = fws[:, O:O + 1]

    v1_col = jax.nn.sigmoid(y1_col)
    v1 = v1_col.reshape(N // 128, 128)
    m1, m1_col, y2 = _sel_matvec(ori_adj, v1, v1_col, s_col, k1)
    v2_col = jax.nn.sigmoid(y2)
    v2 = v2_col.reshape(N // 128, 128)
    m2_col = _topk_mask_col(v2, m1, v2_col, m1_col, k2)

    idx_pad = jnp.concatenate(
        [anchor_nodes_idx.astype(jnp.int32), jnp.zeros((AP - A,), jnp.int32)])

    xa = _sc_gather(features, idx_pad)
    naa = _knn1(features, W1, xa, A)

    pb = jax.nn.one_hot(idx_pad, N, dtype=jnp.bfloat16)
    fr_arr = jnp.asarray(fusion_ratio, jnp.float32).reshape(1, 1)
    out = _refine(naa, fw, nvd, pb, W2, m2_col, m1_col, fr_arr, A)
    return out[:, :A]
